# Initial kernel scaffold; baseline (speedup 1.0000x reference)
import jax
import jax.numpy as jnp
from jax.experimental import pallas as pl


def kernel(z_t, g_t, prototypes, proto_age, m_a, m_a_init):
    raise NotImplementedError("write your pallas kernel here")


# speedup vs baseline: 11.4575x; 11.4575x over previous
import functools

import jax
import jax.numpy as jnp
from jax.experimental import pallas as pl
from jax.experimental.pallas import tpu as pltpu

B = 2048
D = 2048
K = 64
T = 64
NC = B // T
MU = 0.9
RHO_A = 0.99
DELTA = 1.0
ALPHA_P = 1.0
ALPHA_R = 0.5

_HI = jax.lax.Precision.HIGHEST


def _dot11(a, b):
    return jax.lax.dot_general(a, b, (((1,), (1,)), ((), ())),
                               precision=_HI, preferred_element_type=jnp.float32)


def _dot00(a, b):
    return jax.lax.dot_general(a, b, (((0,), (0,)), ((), ())),
                               precision=_HI, preferred_element_type=jnp.float32)


def _vault_kernel(g_sref, init_sref, z_ref, ga_ref, proto_ref, ma_ref,
                  out_ref, P_scr, PG_scr, ma_scr, anom_scr, acc):
    ph = pl.program_id(0)
    c = pl.program_id(1)
    lane = jax.lax.broadcasted_iota(jnp.int32, (1, K), 1)
    sub = jax.lax.broadcasted_iota(jnp.int32, (T, 1), 0)
    eyeK = (jax.lax.broadcasted_iota(jnp.int32, (K, K), 0)
            == jax.lax.broadcasted_iota(jnp.int32, (K, K), 1))

    @pl.when(jnp.logical_and(ph == 0, c == 0))
    def _init():
        P_scr[...] = proto_ref[...]
        anom_scr[...] = jnp.zeros_like(anom_scr)
        acc[0] = 0.0
        acc[1] = 0.0
        acc[2] = 0.0

    Zc = z_ref[...]
    zn_col = jnp.sum(Zc * Zc, axis=1, keepdims=True)

    @pl.when(ph == 0)
    def _scan_phase():
        ga_row = ga_ref[0]
        anom_scr[...] = anom_scr[...] + jnp.dot(
            ga_row, Zc, precision=_HI, preferred_element_type=jnp.float32)
        acc[0] = acc[0] + jnp.sum(ga_row)

        P = P_scr[...]
        H = _dot11(Zc, P)
        G = _dot11(Zc, Zc)
        PP = _dot11(P, P)
        pn0 = jnp.sum(jnp.where(eyeK, PP, 0.0), axis=0, keepdims=True)

        def step(t, carry):
            a_row, a_col, C, pn = carry
            tsc = jnp.where(sub == t, 1.0, 0.0)
            Hrow = _dot00(tsc, H)
            Grow = _dot00(tsc, G)
            znt = _dot00(tsc, zn_col)
            s = a_row * Hrow + jnp.dot(Grow, C, precision=_HI,
                                       preferred_element_type=jnp.float32)
            d2 = znt + pn - 2.0 * s
            m = jnp.min(d2, axis=1, keepdims=True)
            idxv = jnp.where(d2 == m, lane, K)
            kmin = jnp.min(idxv, axis=1, keepdims=True)
            uf = jnp.where(g_sref[c * T + t] > 0, 1.0, 0.0)
            khr = jnp.where(lane == kmin, uf, 0.0)
            khc = jnp.where(sub == kmin, uf, 0.0)
            pn_upd = (MU * MU) * pn + (1.0 - MU) ** 2 * znt \
                + 2.0 * MU * (1.0 - MU) * s
            pn = pn + khr * (pn_upd - pn)
            a_row = a_row * (1.0 - (1.0 - MU) * khr)
            a_col = a_col * (1.0 - (1.0 - MU) * khc)
            C = C * (1.0 - (1.0 - MU) * khr) + (1.0 - MU) * (tsc * khr)
            return a_row, a_col, C, pn

        carry0 = (jnp.ones((1, K), jnp.float32), jnp.ones((T, 1), jnp.float32),
                  jnp.zeros((T, K), jnp.float32), pn0)
        _, a_col, C, _ = jax.lax.fori_loop(0, T, step, carry0)
        P_scr[...] = a_col * P + _dot00(C, Zc)

    @pl.when(ph == 1)
    def _loss_phase():
        @pl.when(c == 0)
        def _finalize_state():
            P = P_scr[...]
            PG_scr[...] = _dot11(P, P)
            cnt = acc[0]
            z_a = anom_scr[...] / jnp.maximum(cnt, 1.0)
            ma_in = ma_ref[...]
            fi = jnp.where(init_sref[0] > 0, 1.0, 0.0)
            cf = jnp.where(cnt > 0.0, 1.0, 0.0)
            ma_upd = fi * (RHO_A * ma_in + (1.0 - RHO_A) * z_a) \
                + (1.0 - fi) * z_a
            new_ma = cf * ma_upd + (1.0 - cf) * ma_in
            ma_scr[...] = new_ma
            acc[3] = jnp.sum(new_ma * new_ma)
            acc[4] = jnp.maximum(fi, cf)

        P = P_scr[...]
        PG = PG_scr[...]
        pn_row = jnp.sum(jnp.where(eyeK, PG, 0.0), axis=0, keepdims=True)
        H2 = _dot11(Zc, P)
        d2 = zn_col + pn_row - 2.0 * H2
        laneTK = jax.lax.broadcasted_iota(jnp.int32, (T, K), 1)
        dcur = d2
        ehot = jnp.zeros((T, K), jnp.float32)
        for _ in range(3):
            m = jnp.min(dcur, axis=1, keepdims=True)
            idxv = jnp.where(dcur == m, laneTK, K)
            kmin = jnp.min(idxv, axis=1, keepdims=True)
            hot = (laneTK == kmin).astype(jnp.float32)
            ehot = ehot + hot
            dcur = jnp.where(hot > 0.0, jnp.float32(3e38), dcur)
        sumHtop = jnp.sum(ehot * H2, axis=1, keepdims=True)
        ePG = jnp.dot(ehot, PG, precision=_HI,
                      preferred_element_type=jnp.float32)
        quad = jnp.sum(ehot * ePG, axis=1, keepdims=True)
        acc[1] = acc[1] + jnp.sum(
            zn_col - (2.0 / 3.0) * sumHtop + (1.0 / 9.0) * quad)

        ma = ma_scr[...]
        zma = _dot11(Zc, ma)
        da2 = jnp.maximum(zn_col - 2.0 * zma + acc[3], 0.0)
        da = jnp.sqrt(da2)
        acc[2] = acc[2] + jnp.sum(jnp.maximum(DELTA - da, 0.0))

        @pl.when(c == NC - 1)
        def _emit():
            l_pull = acc[1] / jnp.float32(B * D)
            l_push = acc[4] * (acc[2] / jnp.float32(B))
            out_ref[0] = ALPHA_P * l_pull + ALPHA_R * l_push


@functools.partial(jax.jit, static_argnames=())
def kernel(z_t, g_t, prototypes, proto_age, m_a, m_a_init):
    del proto_age
    g_i32 = g_t.astype(jnp.int32)
    init_i32 = m_a_init.astype(jnp.int32).reshape((1,))
    ga = (g_t == 0).astype(jnp.float32).reshape(NC, 1, T)
    ma2d = m_a.reshape(1, D)

    grid_spec = pltpu.PrefetchScalarGridSpec(
        num_scalar_prefetch=2,
        grid=(2, NC),
        in_specs=[
            pl.BlockSpec((T, D), lambda ph, c, *_: (c, 0)),
            pl.BlockSpec((1, 1, T), lambda ph, c, *_: (c, 0, 0)),
            pl.BlockSpec((K, D), lambda ph, c, *_: (0, 0)),
            pl.BlockSpec((1, D), lambda ph, c, *_: (0, 0)),
        ],
        out_specs=pl.BlockSpec(memory_space=pltpu.SMEM),
        scratch_shapes=[
            pltpu.VMEM((K, D), jnp.float32),
            pltpu.VMEM((K, K), jnp.float32),
            pltpu.VMEM((1, D), jnp.float32),
            pltpu.VMEM((1, D), jnp.float32),
            pltpu.SMEM((8,), jnp.float32),
        ],
    )
    out = pl.pallas_call(
        _vault_kernel,
        grid_spec=grid_spec,
        out_shape=jax.ShapeDtypeStruct((1,), jnp.float32),
    )(g_i32, init_i32, z_t, ga, prototypes, ma2d)
    return out.reshape(())


# speedup vs baseline: 23.6949x; 2.0681x over previous
import functools

import jax
import jax.numpy as jnp
from jax.experimental import pallas as pl
from jax.experimental.pallas import tpu as pltpu

B = 2048
D = 2048
K = 64
T = 64
NC = B // T
MU = 0.9
RHO_A = 0.99
DELTA = 1.0
ALPHA_P = 1.0
ALPHA_R = 0.5

_HI = jax.lax.Precision.HIGHEST


def _dot11(a, b):
    return jax.lax.dot_general(a, b, (((1,), (1,)), ((), ())),
                               precision=_HI, preferred_element_type=jnp.float32)


def _dot00(a, b):
    return jax.lax.dot_general(a, b, (((0,), (0,)), ((), ())),
                               precision=_HI, preferred_element_type=jnp.float32)


def _vault_kernel(g_sref, init_sref, z_ref, ga_ref, proto_ref, ma_ref,
                  out_ref, P_scr, PG_scr, ma_scr, anom_scr, acc):
    ph = pl.program_id(0)
    c = pl.program_id(1)
    lane = jax.lax.broadcasted_iota(jnp.int32, (1, K), 1)
    sub = jax.lax.broadcasted_iota(jnp.int32, (T, 1), 0)
    eyeK = (jax.lax.broadcasted_iota(jnp.int32, (K, K), 0)
            == jax.lax.broadcasted_iota(jnp.int32, (K, K), 1))

    @pl.when(jnp.logical_and(ph == 0, c == 0))
    def _init():
        P_scr[...] = proto_ref[...]
        anom_scr[...] = jnp.zeros_like(anom_scr)
        acc[0] = 0.0
        acc[1] = 0.0
        acc[2] = 0.0

    Zc = z_ref[...]
    zn_col = jnp.sum(Zc * Zc, axis=1, keepdims=True)

    @pl.when(ph == 0)
    def _scan_phase():
        ga_row = ga_ref[0]
        anom_scr[...] = anom_scr[...] + jnp.dot(
            ga_row, Zc, precision=_HI, preferred_element_type=jnp.float32)
        acc[0] = acc[0] + jnp.sum(ga_row)

        P = P_scr[...]
        H = _dot11(Zc, P)
        G = _dot11(Zc, Zc)
        PP = _dot11(P, P)
        pn0 = jnp.sum(jnp.where(eyeK, PP, 0.0), axis=0, keepdims=True)
        subTK = jax.lax.broadcasted_iota(jnp.int32, (T, K), 0)
        laneTT = jax.lax.broadcasted_iota(jnp.int32, (T, T), 1)

        def step(t, carry):
            Scur, a_col, C, pn = carry
            srow = jnp.sum(jnp.where(subTK == t, Scur, 0.0),
                           axis=0, keepdims=True)
            znt = jnp.sum(jnp.where(sub == t, zn_col, 0.0),
                          axis=0, keepdims=True)
            d2 = znt + pn - 2.0 * srow
            m = jnp.min(d2, axis=1, keepdims=True)
            idxv = jnp.where(d2 == m, lane, K)
            kmin = jnp.min(idxv, axis=1, keepdims=True)
            uf = jnp.where(g_sref[c * T + t] > 0, 1.0, 0.0)
            khr = jnp.where(lane == kmin, uf, 0.0)
            khc = jnp.where(sub == kmin, uf, 0.0)
            tsc = jnp.where(sub == t, 1.0, 0.0)
            Gcol = jnp.sum(jnp.where(laneTT == t, G, 0.0),
                           axis=1, keepdims=True)
            Scur = Scur + khr * ((MU - 1.0) * Scur + (1.0 - MU) * Gcol)
            pn_upd = (MU * MU) * pn + (1.0 - MU) ** 2 * znt \
                + 2.0 * MU * (1.0 - MU) * srow
            pn = pn + khr * (pn_upd - pn)
            a_col = a_col * (1.0 - (1.0 - MU) * khc)
            C = C * (1.0 - (1.0 - MU) * khr) + (1.0 - MU) * (tsc * khr)
            return Scur, a_col, C, pn

        carry0 = (H, jnp.ones((T, 1), jnp.float32),
                  jnp.zeros((T, K), jnp.float32), pn0)
        _, a_col, C, _ = jax.lax.fori_loop(0, T, step, carry0, unroll=4)
        P_scr[...] = a_col * P + _dot00(C, Zc)

    @pl.when(ph == 1)
    def _loss_phase():
        @pl.when(c == 0)
        def _finalize_state():
            P = P_scr[...]
            PG_scr[...] = _dot11(P, P)
            cnt = acc[0]
            z_a = anom_scr[...] / jnp.maximum(cnt, 1.0)
            ma_in = ma_ref[...]
            fi = jnp.where(init_sref[0] > 0, 1.0, 0.0)
            cf = jnp.where(cnt > 0.0, 1.0, 0.0)
            ma_upd = fi * (RHO_A * ma_in + (1.0 - RHO_A) * z_a) \
                + (1.0 - fi) * z_a
            new_ma = cf * ma_upd + (1.0 - cf) * ma_in
            ma_scr[...] = new_ma
            acc[3] = jnp.sum(new_ma * new_ma)
            acc[4] = jnp.maximum(fi, cf)

        P = P_scr[...]
        PG = PG_scr[...]
        pn_row = jnp.sum(jnp.where(eyeK, PG, 0.0), axis=0, keepdims=True)
        H2 = _dot11(Zc, P)
        d2 = zn_col + pn_row - 2.0 * H2
        laneTK = jax.lax.broadcasted_iota(jnp.int32, (T, K), 1)
        dcur = d2
        ehot = jnp.zeros((T, K), jnp.float32)
        for _ in range(3):
            m = jnp.min(dcur, axis=1, keepdims=True)
            idxv = jnp.where(dcur == m, laneTK, K)
            kmin = jnp.min(idxv, axis=1, keepdims=True)
            hot = (laneTK == kmin).astype(jnp.float32)
            ehot = ehot + hot
            dcur = jnp.where(hot > 0.0, jnp.float32(3e38), dcur)
        sumHtop = jnp.sum(ehot * H2, axis=1, keepdims=True)
        ePG = jnp.dot(ehot, PG, precision=_HI,
                      preferred_element_type=jnp.float32)
        quad = jnp.sum(ehot * ePG, axis=1, keepdims=True)
        acc[1] = acc[1] + jnp.sum(
            zn_col - (2.0 / 3.0) * sumHtop + (1.0 / 9.0) * quad)

        ma = ma_scr[...]
        zma = _dot11(Zc, ma)
        da2 = jnp.maximum(zn_col - 2.0 * zma + acc[3], 0.0)
        da = jnp.sqrt(da2)
        acc[2] = acc[2] + jnp.sum(jnp.maximum(DELTA - da, 0.0))

        @pl.when(c == NC - 1)
        def _emit():
            l_pull = acc[1] / jnp.float32(B * D)
            l_push = acc[4] * (acc[2] / jnp.float32(B))
            out_ref[0] = ALPHA_P * l_pull + ALPHA_R * l_push


@functools.partial(jax.jit, static_argnames=())
def kernel(z_t, g_t, prototypes, proto_age, m_a, m_a_init):
    del proto_age
    g_i32 = g_t.astype(jnp.int32)
    init_i32 = m_a_init.astype(jnp.int32).reshape((1,))
    ga = (g_t == 0).astype(jnp.float32).reshape(NC, 1, T)
    ma2d = m_a.reshape(1, D)

    grid_spec = pltpu.PrefetchScalarGridSpec(
        num_scalar_prefetch=2,
        grid=(2, NC),
        in_specs=[
            pl.BlockSpec((T, D), lambda ph, c, *_: (c, 0)),
            pl.BlockSpec((1, 1, T), lambda ph, c, *_: (c, 0, 0)),
            pl.BlockSpec((K, D), lambda ph, c, *_: (0, 0)),
            pl.BlockSpec((1, D), lambda ph, c, *_: (0, 0)),
        ],
        out_specs=pl.BlockSpec(memory_space=pltpu.SMEM),
        scratch_shapes=[
            pltpu.VMEM((K, D), jnp.float32),
            pltpu.VMEM((K, K), jnp.float32),
            pltpu.VMEM((1, D), jnp.float32),
            pltpu.VMEM((1, D), jnp.float32),
            pltpu.SMEM((8,), jnp.float32),
        ],
    )
    out = pl.pallas_call(
        _vault_kernel,
        grid_spec=grid_spec,
        out_shape=jax.ShapeDtypeStruct((1,), jnp.float32),
    )(g_i32, init_i32, z_t, ga, prototypes, ma2d)
    return out.reshape(())


# speedup vs baseline: 23.7881x; 1.0039x over previous
import jax
import jax.numpy as jnp
from jax.experimental import pallas as pl
from jax.experimental.pallas import tpu as pltpu

B = 2048
D = 2048
K = 64
T = 64
NC = B // T
MU = 0.9
RHO_A = 0.99
DELTA = 1.0
ALPHA_P = 1.0
ALPHA_R = 0.5

_HI = jax.lax.Precision.HIGHEST


def _dot11(a, b):
    return jax.lax.dot_general(a, b, (((1,), (1,)), ((), ())),
                               precision=_HI, preferred_element_type=jnp.float32)


def _dot00(a, b):
    return jax.lax.dot_general(a, b, (((0,), (0,)), ((), ())),
                               precision=_HI, preferred_element_type=jnp.float32)


def _vault_kernel(g_sref, init_sref, z_ref, ga_ref, proto_ref, ma_ref,
                  out_ref, P_scr):
    lane = jax.lax.broadcasted_iota(jnp.int32, (1, K), 1)
    sub = jax.lax.broadcasted_iota(jnp.int32, (T, 1), 0)
    eyeK = (jax.lax.broadcasted_iota(jnp.int32, (K, K), 0)
            == jax.lax.broadcasted_iota(jnp.int32, (K, K), 1))
    subTK = jax.lax.broadcasted_iota(jnp.int32, (T, K), 0)
    laneTT = jax.lax.broadcasted_iota(jnp.int32, (T, T), 1)

    P_scr[...] = proto_ref[...]

    def chunk_body(c, _):
        Zc = z_ref[pl.ds(c * T, T), :]
        zn_col = jnp.sum(Zc * Zc, axis=1, keepdims=True)
        P = P_scr[...]
        H = _dot11(Zc, P)
        G = _dot11(Zc, Zc)
        PP = _dot11(P, P)
        pn0 = jnp.sum(jnp.where(eyeK, PP, 0.0), axis=0, keepdims=True)

        def step(t, carry):
            Scur, a_col, C, pn = carry
            srow = jnp.sum(jnp.where(subTK == t, Scur, 0.0),
                           axis=0, keepdims=True)
            znt = jnp.sum(jnp.where(sub == t, zn_col, 0.0),
                          axis=0, keepdims=True)
            d2 = znt + pn - 2.0 * srow
            m = jnp.min(d2, axis=1, keepdims=True)
            idxv = jnp.where(d2 == m, lane, K)
            kmin = jnp.min(idxv, axis=1, keepdims=True)
            uf = jnp.where(g_sref[c * T + t] > 0, 1.0, 0.0)
            khr = jnp.where(lane == kmin, uf, 0.0)
            khc = jnp.where(sub == kmin, uf, 0.0)
            tsc = jnp.where(sub == t, 1.0, 0.0)
            Gcol = jnp.sum(jnp.where(laneTT == t, G, 0.0),
                           axis=1, keepdims=True)
            Scur = Scur + khr * ((MU - 1.0) * Scur + (1.0 - MU) * Gcol)
            pn_upd = (MU * MU) * pn + (1.0 - MU) ** 2 * znt \
                + 2.0 * MU * (1.0 - MU) * srow
            pn = pn + khr * (pn_upd - pn)
            a_col = a_col * (1.0 - (1.0 - MU) * khc)
            C = C * (1.0 - (1.0 - MU) * khr) + (1.0 - MU) * (tsc * khr)
            return Scur, a_col, C, pn

        carry0 = (H, jnp.ones((T, 1), jnp.float32),
                  jnp.zeros((T, K), jnp.float32), pn0)
        _, a_col, C, _ = jax.lax.fori_loop(0, T, step, carry0, unroll=4)
        P_scr[...] = a_col * P + _dot00(C, Zc)
        return 0

    jax.lax.fori_loop(0, NC, chunk_body, 0)

    ga_row = ga_ref[...]
    Zfull = z_ref[...]
    anom = jax.lax.dot_general(ga_row, Zfull, (((1,), (0,)), ((), ())),
                               precision=_HI,
                               preferred_element_type=jnp.float32)
    cnt = jnp.sum(ga_row)
    z_a = anom / jnp.maximum(cnt, 1.0)
    ma_in = ma_ref[...]
    fi = jnp.where(init_sref[0] > 0, 1.0, 0.0)
    cf = jnp.where(cnt > 0.0, 1.0, 0.0)
    ma_upd = fi * (RHO_A * ma_in + (1.0 - RHO_A) * z_a) + (1.0 - fi) * z_a
    new_ma = cf * ma_upd + (1.0 - cf) * ma_in
    man2 = jnp.sum(new_ma * new_ma)
    new_init = jnp.maximum(fi, cf)

    Pf = P_scr[...]
    PG = _dot11(Pf, Pf)
    pn_row = jnp.sum(jnp.where(eyeK, PG, 0.0), axis=0, keepdims=True)
    laneTK = jax.lax.broadcasted_iota(jnp.int32, (T, K), 1)

    def loss_body(c, accs):
        pull_acc, push_acc = accs
        Zc = z_ref[pl.ds(c * T, T), :]
        zn_col = jnp.sum(Zc * Zc, axis=1, keepdims=True)
        H2 = _dot11(Zc, Pf)
        d2 = zn_col + pn_row - 2.0 * H2
        dcur = d2
        ehot = jnp.zeros((T, K), jnp.float32)
        for _ in range(3):
            m = jnp.min(dcur, axis=1, keepdims=True)
            idxv = jnp.where(dcur == m, laneTK, K)
            kmin = jnp.min(idxv, axis=1, keepdims=True)
            hot = (laneTK == kmin).astype(jnp.float32)
            ehot = ehot + hot
            dcur = jnp.where(hot > 0.0, jnp.float32(3e38), dcur)
        sumHtop = jnp.sum(ehot * H2, axis=1, keepdims=True)
        ePG = jnp.dot(ehot, PG, precision=_HI,
                      preferred_element_type=jnp.float32)
        quad = jnp.sum(ehot * ePG, axis=1, keepdims=True)
        pull_acc = pull_acc + jnp.sum(
            zn_col - (2.0 / 3.0) * sumHtop + (1.0 / 9.0) * quad)
        zma = _dot11(Zc, new_ma)
        da = jnp.sqrt(jnp.maximum(zn_col - 2.0 * zma + man2, 0.0))
        push_acc = push_acc + jnp.sum(jnp.maximum(DELTA - da, 0.0))
        return pull_acc, push_acc

    pull_acc, push_acc = jax.lax.fori_loop(
        0, NC, loss_body, (jnp.float32(0.0), jnp.float32(0.0)))

    l_pull = pull_acc / jnp.float32(B * D)
    l_push = new_init * (push_acc / jnp.float32(B))
    out_ref[0] = ALPHA_P * l_pull + ALPHA_R * l_push


@jax.jit
def kernel(z_t, g_t, prototypes, proto_age, m_a, m_a_init):
    del proto_age
    g_i32 = g_t.astype(jnp.int32)
    init_i32 = m_a_init.astype(jnp.int32).reshape((1,))
    ga = (g_t == 0).astype(jnp.float32).reshape(1, B)
    ma2d = m_a.reshape(1, D)

    grid_spec = pltpu.PrefetchScalarGridSpec(
        num_scalar_prefetch=2,
        grid=(1,),
        in_specs=[
            pl.BlockSpec((B, D), lambda i, *_: (0, 0)),
            pl.BlockSpec((1, B), lambda i, *_: (0, 0)),
            pl.BlockSpec((K, D), lambda i, *_: (0, 0)),
            pl.BlockSpec((1, D), lambda i, *_: (0, 0)),
        ],
        out_specs=pl.BlockSpec(memory_space=pltpu.SMEM),
        scratch_shapes=[
            pltpu.VMEM((K, D), jnp.float32),
        ],
    )
    out = pl.pallas_call(
        _vault_kernel,
        grid_spec=grid_spec,
        out_shape=jax.ShapeDtypeStruct((1,), jnp.float32),
    )(g_i32, init_i32, z_t, ga, prototypes, ma2d)
    return out.reshape(())


# speedup vs baseline: 36.4388x; 1.5318x over previous
import jax
import jax.numpy as jnp
from jax.experimental import pallas as pl
from jax.experimental.pallas import tpu as pltpu

B = 2048
D = 2048
K = 64
T = 64
NC = B // T
MU = 0.9
RHO_A = 0.99
DELTA = 1.0
ALPHA_P = 1.0
ALPHA_R = 0.5

_HI = jax.lax.Precision.HIGHEST


def _dot11(a, b):
    return jax.lax.dot_general(a, b, (((1,), (1,)), ((), ())),
                               precision=_HI, preferred_element_type=jnp.float32)


def _dot00(a, b):
    return jax.lax.dot_general(a, b, (((0,), (0,)), ((), ())),
                               precision=_HI, preferred_element_type=jnp.float32)


def _vault_kernel(g_sref, init_sref, z_ref, ga_ref, proto_ref, ma_ref,
                  out_ref, P_scr):
    lane = jax.lax.broadcasted_iota(jnp.int32, (1, K), 1)
    sub = jax.lax.broadcasted_iota(jnp.int32, (T, 1), 0)
    eyeK = (jax.lax.broadcasted_iota(jnp.int32, (K, K), 0)
            == jax.lax.broadcasted_iota(jnp.int32, (K, K), 1))
    subTK = jax.lax.broadcasted_iota(jnp.int32, (T, K), 0)
    laneTT = jax.lax.broadcasted_iota(jnp.int32, (T, T), 1)

    P_scr[...] = proto_ref[...]

    def chunk_body(c, _):
        Zc = z_ref[pl.ds(c * T, T), :]
        zn_col = jnp.sum(Zc * Zc, axis=1, keepdims=True)
        P = P_scr[...]
        H = _dot11(Zc, P)
        G = _dot11(Zc, Zc)
        PP = _dot11(P, P)
        pn0 = jnp.sum(jnp.where(eyeK, PP, 0.0), axis=0, keepdims=True)

        def step(t, carry):
            def active(cr):
                Scur, a_col, C, pn = cr
                srow = jnp.sum(jnp.where(subTK == t, Scur, 0.0),
                               axis=0, keepdims=True)
                znt = jnp.sum(jnp.where(sub == t, zn_col, 0.0),
                              axis=0, keepdims=True)
                d2 = znt + pn - 2.0 * srow
                m = jnp.min(d2, axis=1, keepdims=True)
                idxv = jnp.where(d2 == m, lane, K)
                kmin = jnp.min(idxv, axis=1, keepdims=True)
                khr = jnp.where(lane == kmin, 1.0, 0.0)
                khc = jnp.where(sub == kmin, 1.0, 0.0)
                tsc = jnp.where(sub == t, 1.0, 0.0)
                Gcol = jnp.sum(jnp.where(laneTT == t, G, 0.0),
                               axis=1, keepdims=True)
                Scur = Scur + khr * ((MU - 1.0) * Scur + (1.0 - MU) * Gcol)
                pn_upd = (MU * MU) * pn + (1.0 - MU) ** 2 * znt \
                    + 2.0 * MU * (1.0 - MU) * srow
                pn = pn + khr * (pn_upd - pn)
                a_col = a_col * (1.0 - (1.0 - MU) * khc)
                C = C * (1.0 - (1.0 - MU) * khr) + (1.0 - MU) * (tsc * khr)
                return Scur, a_col, C, pn

            return jax.lax.cond(g_sref[c * T + t] > 0, active,
                                lambda cr: cr, carry)

        carry0 = (H, jnp.ones((T, 1), jnp.float32),
                  jnp.zeros((T, K), jnp.float32), pn0)
        _, a_col, C, _ = jax.lax.fori_loop(0, T, step, carry0, unroll=4)
        P_scr[...] = a_col * P + _dot00(C, Zc)
        return 0

    jax.lax.fori_loop(0, NC, chunk_body, 0)

    ga_row = ga_ref[...]
    Zfull = z_ref[...]
    anom = jax.lax.dot_general(ga_row, Zfull, (((1,), (0,)), ((), ())),
                               precision=_HI,
                               preferred_element_type=jnp.float32)
    cnt = jnp.sum(ga_row)
    z_a = anom / jnp.maximum(cnt, 1.0)
    ma_in = ma_ref[...]
    fi = jnp.where(init_sref[0] > 0, 1.0, 0.0)
    cf = jnp.where(cnt > 0.0, 1.0, 0.0)
    ma_upd = fi * (RHO_A * ma_in + (1.0 - RHO_A) * z_a) + (1.0 - fi) * z_a
    new_ma = cf * ma_upd + (1.0 - cf) * ma_in
    man2 = jnp.sum(new_ma * new_ma)
    new_init = jnp.maximum(fi, cf)

    Pf = P_scr[...]
    PG = _dot11(Pf, Pf)
    pn_row = jnp.sum(jnp.where(eyeK, PG, 0.0), axis=0, keepdims=True)
    laneTK = jax.lax.broadcasted_iota(jnp.int32, (T, K), 1)

    def loss_body(c, accs):
        pull_acc, push_acc = accs
        Zc = z_ref[pl.ds(c * T, T), :]
        zn_col = jnp.sum(Zc * Zc, axis=1, keepdims=True)
        H2 = _dot11(Zc, Pf)
        d2 = zn_col + pn_row - 2.0 * H2
        dcur = d2
        ehot = jnp.zeros((T, K), jnp.float32)
        for _ in range(3):
            m = jnp.min(dcur, axis=1, keepdims=True)
            idxv = jnp.where(dcur == m, laneTK, K)
            kmin = jnp.min(idxv, axis=1, keepdims=True)
            hot = (laneTK == kmin).astype(jnp.float32)
            ehot = ehot + hot
            dcur = jnp.where(hot > 0.0, jnp.float32(3e38), dcur)
        sumHtop = jnp.sum(ehot * H2, axis=1, keepdims=True)
        ePG = jnp.dot(ehot, PG, precision=_HI,
                      preferred_element_type=jnp.float32)
        quad = jnp.sum(ehot * ePG, axis=1, keepdims=True)
        pull_acc = pull_acc + jnp.sum(
            zn_col - (2.0 / 3.0) * sumHtop + (1.0 / 9.0) * quad)
        zma = _dot11(Zc, new_ma)
        da = jnp.sqrt(jnp.maximum(zn_col - 2.0 * zma + man2, 0.0))
        push_acc = push_acc + jnp.sum(jnp.maximum(DELTA - da, 0.0))
        return pull_acc, push_acc

    pull_acc, push_acc = jax.lax.fori_loop(
        0, NC, loss_body, (jnp.float32(0.0), jnp.float32(0.0)))

    l_pull = pull_acc / jnp.float32(B * D)
    l_push = new_init * (push_acc / jnp.float32(B))
    out_ref[0] = ALPHA_P * l_pull + ALPHA_R * l_push


@jax.jit
def kernel(z_t, g_t, prototypes, proto_age, m_a, m_a_init):
    del proto_age
    g_i32 = g_t.astype(jnp.int32)
    init_i32 = m_a_init.astype(jnp.int32).reshape((1,))
    ga = (g_t == 0).astype(jnp.float32).reshape(1, B)
    ma2d = m_a.reshape(1, D)

    grid_spec = pltpu.PrefetchScalarGridSpec(
        num_scalar_prefetch=2,
        grid=(1,),
        in_specs=[
            pl.BlockSpec((B, D), lambda i, *_: (0, 0)),
            pl.BlockSpec((1, B), lambda i, *_: (0, 0)),
            pl.BlockSpec((K, D), lambda i, *_: (0, 0)),
            pl.BlockSpec((1, D), lambda i, *_: (0, 0)),
        ],
        out_specs=pl.BlockSpec(memory_space=pltpu.SMEM),
        scratch_shapes=[
            pltpu.VMEM((K, D), jnp.float32),
        ],
    )
    out = pl.pallas_call(
        _vault_kernel,
        grid_spec=grid_spec,
        out_shape=jax.ShapeDtypeStruct((1,), jnp.float32),
    )(g_i32, init_i32, z_t, ga, prototypes, ma2d)
    return out.reshape(())
